# Initial kernel scaffold; baseline (speedup 1.0000x reference)
import jax
import jax.numpy as jnp
from jax.experimental import pallas as pl


def kernel(time, value, var_id, category_mask, W1_t, b1_t, W2_t, W1_v, b1_v, W2_v, emb_table):
    raise NotImplementedError("write your pallas kernel here")


# speedup vs baseline: 1.1801x; 1.1801x over previous
import functools

import jax
import jax.numpy as jnp
from jax import lax
from jax.experimental import pallas as pl
from jax.experimental.pallas import tpu as pltpu
from jax.experimental.pallas import tpu_sc as plsc

_EMB = 64
_HID = 8

_NW = 32
_K = 8
_CHUNK = _K * 128


def _sc_gather(emb_table, idx_flat):
    n = idx_flat.shape[0]
    per_w = n // _NW
    steps = per_w // _CHUNK
    idx2 = idx_flat.reshape(n // 128, 128)
    mesh = plsc.VectorSubcoreMesh(core_axis_name="c", subcore_axis_name="s")

    @functools.partial(
        pl.kernel,
        mesh=mesh,
        out_type=jax.ShapeDtypeStruct((n, _EMB), jnp.float32),
        compiler_params=pltpu.CompilerParams(use_tc_tiling_on_sc=False),
        scratch_types=[
            pltpu.VMEM((_K, 128), jnp.int32),
            pltpu.VMEM((_CHUNK, _EMB), jnp.float32),
            pltpu.SemaphoreType.DMA,
        ],
    )
    def gather_kernel(table_hbm, idx_hbm, out_hbm, idx_v, rows_v, sem):
        wid = lax.axis_index("s") * 2 + lax.axis_index("c")
        base = wid * per_w

        def body(i, carry):
            tok0 = pl.multiple_of(base + i * _CHUNK, _CHUNK)
            row0 = pl.multiple_of(tok0 // 128, _K)
            pltpu.sync_copy(idx_hbm.at[pl.ds(row0, _K)], idx_v)
            copies = []
            for j in range(_K):
                copies.append(
                    pltpu.async_copy(
                        table_hbm.at[idx_v.at[j]],
                        rows_v.at[pl.ds(j * 128, 128)],
                        sem,
                    )
                )
            for cp in copies:
                cp.wait()
            pltpu.sync_copy(rows_v, out_hbm.at[pl.ds(tok0, _CHUNK)])
            return carry

        lax.fori_loop(0, steps, body, 0)

    return gather_kernel(emb_table, idx2)


def _tc_body(t_ref, v_ref, id_ref, m_ref, g_ref, w1t, b1t, w2t, w1v, b1v,
             w2v, out_ref, pm_ref):
    ht = jnp.tanh(t_ref[...] * w1t[...] + b1t[...])
    hv = jnp.tanh(v_ref[...] * w1v[...] + b1v[...])
    et = jnp.dot(ht, w2t[...], preferred_element_type=jnp.float32)
    ev = jnp.dot(hv, w2v[...], preferred_element_type=jnp.float32)
    cm = m_ref[...].astype(jnp.float32)
    out_ref[...] = et + ev * cm + g_ref[...]
    pm_ref[...] = jnp.clip(id_ref[...].astype(jnp.float32), 0.0, 1.0)


def _tc_combine(time_c, value_c, id_c, mask_c, gath, W1_t, b1_t, W2_t, W1_v,
                b1_v, W2_v):
    n = time_c.shape[0]
    tok = 1024
    grid = (n // tok,)
    col = lambda i: (i, 0)
    fixed = lambda i: (0, 0)
    return pl.pallas_call(
        _tc_body,
        grid=grid,
        in_specs=[
            pl.BlockSpec((tok, 1), col),
            pl.BlockSpec((tok, 1), col),
            pl.BlockSpec((tok, 1), col),
            pl.BlockSpec((tok, 1), col),
            pl.BlockSpec((tok, _EMB), col),
            pl.BlockSpec((1, _HID), fixed),
            pl.BlockSpec((1, _HID), fixed),
            pl.BlockSpec((_HID, _EMB), fixed),
            pl.BlockSpec((1, _HID), fixed),
            pl.BlockSpec((1, _HID), fixed),
            pl.BlockSpec((_HID, _EMB), fixed),
        ],
        out_specs=[
            pl.BlockSpec((tok, _EMB), col),
            pl.BlockSpec((tok, 1), col),
        ],
        out_shape=[
            jax.ShapeDtypeStruct((n, _EMB), jnp.float32),
            jax.ShapeDtypeStruct((n, 1), jnp.float32),
        ],
    )(time_c, value_c, id_c, mask_c, gath, W1_t, b1_t, W2_t, W1_v, b1_v, W2_v)


def kernel(time, value, var_id, category_mask, W1_t, b1_t, W2_t, W1_v, b1_v,
           W2_v, emb_table):
    b, l = time.shape
    n = b * l
    idx = var_id.reshape(n).astype(jnp.int32)
    gath = _sc_gather(emb_table, idx)
    out, pm = _tc_combine(
        time.reshape(n, 1),
        value.reshape(n, 1),
        idx.reshape(n, 1),
        category_mask.reshape(n, 1).astype(jnp.int32),
        gath,
        W1_t,
        b1_t.reshape(1, _HID),
        W2_t,
        W1_v,
        b1_v.reshape(1, _HID),
        W2_v,
    )
    return out.reshape(b, l, _EMB), pm.reshape(b, l)


# speedup vs baseline: 1.5096x; 1.2792x over previous
import functools

import jax
import jax.numpy as jnp
from jax import lax
from jax.experimental import pallas as pl
from jax.experimental.pallas import tpu as pltpu
from jax.experimental.pallas import tpu_sc as plsc

_EMB = 64
_HID = 8

_NW = 32
_KP = 4
_CHP = _KP * 128


def _sc_gather_pairs(emb_table, idx_e, idx_o, p):
    per_w = p // _NW
    steps = per_w // _CHP
    mesh = plsc.VectorSubcoreMesh(core_axis_name="c", subcore_axis_name="s")

    @functools.partial(
        pl.kernel,
        mesh=mesh,
        out_type=jax.ShapeDtypeStruct((p, 2 * _EMB), jnp.float32),
        scratch_types=[
            pltpu.VMEM((_KP, 128), jnp.int32),
            pltpu.VMEM((_KP, 128), jnp.int32),
            pltpu.VMEM((_CHP, _EMB), jnp.float32),
            pltpu.VMEM((_CHP, _EMB), jnp.float32),
            pltpu.SemaphoreType.DMA,
        ],
        compiler_params=pltpu.CompilerParams(use_tc_tiling_on_sc=False),
    )
    def gather_kernel(table_hbm, idxe_hbm, idxo_hbm, out_hbm, idxe_v, idxo_v,
                      re_v, ro_v, sem):
        wid = lax.axis_index("s") * 2 + lax.axis_index("c")
        base = wid * per_w

        def body(i, carry):
            p0 = pl.multiple_of(base + i * _CHP, _CHP)
            r0 = pl.multiple_of(p0 // 128, _KP)
            pltpu.sync_copy(idxe_hbm.at[pl.ds(r0, _KP)], idxe_v)
            pltpu.sync_copy(idxo_hbm.at[pl.ds(r0, _KP)], idxo_v)
            copies = []
            for j in range(_KP):
                dst = pl.ds(j * 128, 128)
                copies.append(
                    pltpu.async_copy(table_hbm.at[idxe_v.at[j]],
                                     re_v.at[dst], sem))
                copies.append(
                    pltpu.async_copy(table_hbm.at[idxo_v.at[j]],
                                     ro_v.at[dst], sem))
            for cp in copies:
                cp.wait()
            rows = pl.ds(p0, _CHP)
            pltpu.sync_copy(re_v, out_hbm.at[rows, pl.ds(0, _EMB)])
            pltpu.sync_copy(ro_v, out_hbm.at[rows, pl.ds(_EMB, _EMB)])
            return carry

        lax.fori_loop(0, steps, body, 0)

    return gather_kernel(emb_table, idx_e, idx_o)


def _tc_body(t_ref, v_ref, id_ref, m_ref, g_ref, w1t, b1t, w1v, b1v, wp,
             out_ref, pm_ref):
    w = wp[...]
    for r in range(8):
        row = pl.ds(r, 1)
        xt = t_ref[row, :]
        xv = v_ref[row, :]
        cm = m_ref[row, :].astype(jnp.float32)
        ht = jnp.tanh(w1t[...] * xt + b1t[...])
        hv = jnp.tanh(w1v[...] * xv + b1v[...]) * cm
        h = jnp.concatenate([ht, hv], axis=0)
        h32 = jnp.concatenate([h[:, : _EMB], h[:, _EMB:]], axis=0)
        o = lax.dot_general(h32, w, (((0,), (0,)), ((), ())),
                            preferred_element_type=jnp.float32)
        sl = pl.ds(r * _EMB, _EMB)
        out_ref[sl, :] = o + g_ref[sl, :]
    pm_ref[...] = jnp.clip(id_ref[...].astype(jnp.float32), 0.0, 1.0)


def _tc_combine(t2, v2, id2, m2, gath, w1t, b1t, w1v, b1v, w2c):
    rows = t2.shape[0]
    grid = (rows // 8,)
    tile = lambda i: (i, 0)
    fixed = lambda i: (0, 0)
    return pl.pallas_call(
        _tc_body,
        grid=grid,
        in_specs=[
            pl.BlockSpec((8, 128), tile),
            pl.BlockSpec((8, 128), tile),
            pl.BlockSpec((8, 128), tile),
            pl.BlockSpec((8, 128), tile),
            pl.BlockSpec((512, 2 * _EMB), tile),
            pl.BlockSpec((_HID, 1), fixed),
            pl.BlockSpec((_HID, 1), fixed),
            pl.BlockSpec((_HID, 1), fixed),
            pl.BlockSpec((_HID, 1), fixed),
            pl.BlockSpec((4 * _HID, 2 * _EMB), fixed),
        ],
        out_specs=[
            pl.BlockSpec((512, 2 * _EMB), tile),
            pl.BlockSpec((8, 128), tile),
        ],
        out_shape=[
            jax.ShapeDtypeStruct((rows * 64, 2 * _EMB), jnp.float32),
            jax.ShapeDtypeStruct((rows, 128), jnp.float32),
        ],
    )(t2, v2, id2, m2, gath, w1t, b1t, w1v, b1v, w2c)


def kernel(time, value, var_id, category_mask, W1_t, b1_t, W2_t, W1_v, b1_v,
           W2_v, emb_table):
    b, l = time.shape
    n = b * l
    p = n // 2
    idx = var_id.reshape(n).astype(jnp.int32)
    idx_e = idx[0::2].reshape(p // 128, 128)
    idx_o = idx[1::2].reshape(p // 128, 128)
    gath = _sc_gather_pairs(emb_table, idx_e, idx_o, p)

    def perm(x):
        return x.reshape(n // 128, 64, 2).transpose(0, 2, 1).reshape(
            n // 128, 128)

    w2cat = jnp.concatenate([W2_t, W2_v], axis=0)
    zero = jnp.zeros_like(w2cat)
    wp = jnp.concatenate(
        [jnp.concatenate([w2cat, zero], axis=1),
         jnp.concatenate([zero, w2cat], axis=1)], axis=0)

    out, pm = _tc_combine(
        perm(time),
        perm(value),
        var_id.reshape(n // 128, 128).astype(jnp.int32),
        perm(category_mask.astype(jnp.int32)),
        gath,
        W1_t.reshape(_HID, 1),
        b1_t.reshape(_HID, 1),
        W1_v.reshape(_HID, 1),
        b1_v.reshape(_HID, 1),
        wp,
    )
    return out.reshape(b, l, _EMB), pm.reshape(b, l)


# speedup vs baseline: 3.0740x; 2.0363x over previous
import functools

import jax
import jax.numpy as jnp
from jax import lax
from jax.experimental import pallas as pl
from jax.experimental.pallas import tpu as pltpu
from jax.experimental.pallas import tpu_sc as plsc

_EMB = 64
_HID = 8

_NW = 32
_KP = 4
_CHP = _KP * 128


def _sc_gather_pairs(emb_table, idx_e, idx_o, p):
    per_w = p // _NW
    steps = per_w // _CHP
    mesh = plsc.VectorSubcoreMesh(core_axis_name="c", subcore_axis_name="s")

    @functools.partial(
        pl.kernel,
        mesh=mesh,
        out_type=jax.ShapeDtypeStruct((p, 2 * _EMB), jnp.float32),
        scratch_types=[
            pltpu.VMEM((_KP, 128), jnp.int32),
            pltpu.VMEM((_KP, 128), jnp.int32),
            pltpu.VMEM((_CHP, _EMB), jnp.float32),
            pltpu.VMEM((_CHP, _EMB), jnp.float32),
            pltpu.SemaphoreType.DMA,
        ],
        compiler_params=pltpu.CompilerParams(use_tc_tiling_on_sc=False),
    )
    def gather_kernel(table_hbm, idxe_hbm, idxo_hbm, out_hbm, idxe_v, idxo_v,
                      re_v, ro_v, sem):
        wid = lax.axis_index("s") * 2 + lax.axis_index("c")
        base = wid * per_w

        def body(i, carry):
            p0 = pl.multiple_of(base + i * _CHP, _CHP)
            r0 = pl.multiple_of(p0 // 128, _KP)
            pltpu.sync_copy(idxe_hbm.at[pl.ds(r0, _KP)], idxe_v)
            pltpu.sync_copy(idxo_hbm.at[pl.ds(r0, _KP)], idxo_v)
            copies = []
            for j in range(_KP):
                dst = pl.ds(j * 128, 128)
                copies.append(
                    pltpu.async_copy(table_hbm.at[idxe_v.at[j]],
                                     re_v.at[dst], sem))
                copies.append(
                    pltpu.async_copy(table_hbm.at[idxo_v.at[j]],
                                     ro_v.at[dst], sem))
            for cp in copies:
                cp.wait()
            rows = pl.ds(p0, _CHP)
            pltpu.sync_copy(re_v, out_hbm.at[rows, pl.ds(0, _EMB)])
            pltpu.sync_copy(ro_v, out_hbm.at[rows, pl.ds(_EMB, _EMB)])
            return carry

        lax.fori_loop(0, steps, body, 0)

    return gather_kernel(emb_table, idx_e, idx_o)


def _tc_body(t_ref, v_ref, m_ref, g_ref, w1t, b1t, w1v, b1v, w2c, me_r, mo_r,
             out_ref):
    w2 = w2c[...]
    me = me_r[...]
    mo = mo_r[...]
    dn = (((0,), (0,)), ((), ()))
    for k in range(8):
        row = pl.ds(k, 1)
        xt = t_ref[0, row, :]
        xv = v_ref[0, row, :]
        cm = m_ref[0, row, :].astype(jnp.float32)
        ht = jnp.tanh(w1t[...] * xt + b1t[...])
        hv = jnp.tanh(w1v[...] * xv + b1v[...]) * cm
        h = jnp.concatenate([ht, hv], axis=0)
        cve = lax.dot_general(w2, h, dn,
                              preferred_element_type=jnp.float32)
        pr = g_ref[pl.ds(k * _EMB, _EMB), :]
        ge = lax.dot_general(pr[:, : _EMB], me, dn,
                             preferred_element_type=jnp.float32)
        go = lax.dot_general(pr[:, _EMB:], mo, dn,
                             preferred_element_type=jnp.float32)
        out_ref[0, :, pl.ds(k * 128, 128)] = cve + ge + go


def _tc_combine(t3, v3, m3, gath, w1t, b1t, w1v, b1v, w2c, me, mo, l, b):
    bt = b // 1024
    return pl.pallas_call(
        _tc_body,
        grid=(l, bt),
        in_specs=[
            pl.BlockSpec((1, 8, 128), lambda i, j: (i, j, 0)),
            pl.BlockSpec((1, 8, 128), lambda i, j: (i, j, 0)),
            pl.BlockSpec((1, 8, 128), lambda i, j: (i, j, 0)),
            pl.BlockSpec((512, 128), lambda i, j: (i * 4 + j, 0)),
            pl.BlockSpec((_HID, 1), lambda i, j: (0, 0)),
            pl.BlockSpec((_HID, 1), lambda i, j: (0, 0)),
            pl.BlockSpec((_HID, 1), lambda i, j: (0, 0)),
            pl.BlockSpec((_HID, 1), lambda i, j: (0, 0)),
            pl.BlockSpec((2 * _HID, _EMB), lambda i, j: (0, 0)),
            pl.BlockSpec((_EMB, 128), lambda i, j: (0, 0)),
            pl.BlockSpec((_EMB, 128), lambda i, j: (0, 0)),
        ],
        out_specs=pl.BlockSpec((1, _EMB, 1024), lambda i, j: (i, 0, j)),
        out_shape=jax.ShapeDtypeStruct((l, _EMB, b), jnp.float32),
    )(t3, v3, m3, gath, w1t, b1t, w1v, b1v, w2c, me, mo)


def _pm_body(id_ref, pm_ref):
    pm_ref[...] = jnp.clip(id_ref[...].astype(jnp.float32), 0.0, 1.0)


def _pm_mask(id_t, l, b):
    return pl.pallas_call(
        _pm_body,
        grid=(l // 8,),
        in_specs=[pl.BlockSpec((8, b), lambda i: (i, 0))],
        out_specs=pl.BlockSpec((8, b), lambda i: (i, 0)),
        out_shape=jax.ShapeDtypeStruct((l, b), jnp.float32),
    )(id_t)


def kernel(time, value, var_id, category_mask, W1_t, b1_t, W2_t, W1_v, b1_v,
           W2_v, emb_table):
    b, l = time.shape
    n = b * l
    p = n // 2

    id_t = var_id.T.astype(jnp.int32)
    idx = id_t.reshape(n)
    idx_e = idx[0::2].reshape(p // 128, 128)
    idx_o = idx[1::2].reshape(p // 128, 128)
    gath = _sc_gather_pairs(emb_table, idx_e, idx_o, p)

    cols = jnp.arange(128, dtype=jnp.int32)[None, :]
    rows = jnp.arange(_EMB, dtype=jnp.int32)[:, None]
    me = (cols == 2 * rows).astype(jnp.float32)
    mo = (cols == 2 * rows + 1).astype(jnp.float32)

    out3 = _tc_combine(
        time.T.reshape(l, b // 128, 128),
        value.T.reshape(l, b // 128, 128),
        category_mask.T.reshape(l, b // 128, 128).astype(jnp.int32),
        gath,
        W1_t.reshape(_HID, 1),
        b1_t.reshape(_HID, 1),
        W1_v.reshape(_HID, 1),
        b1_v.reshape(_HID, 1),
        jnp.concatenate([W2_t, W2_v], axis=0),
        me,
        mo,
        l,
        b,
    )
    pm = _pm_mask(id_t, l, b)
    return out3.transpose(2, 0, 1), pm.T


# speedup vs baseline: 4.0800x; 1.3273x over previous
import functools

import jax
import jax.numpy as jnp
from jax import lax
from jax.experimental import pallas as pl
from jax.experimental.pallas import tpu as pltpu
from jax.experimental.pallas import tpu_sc as plsc

_EMB = 64
_HID = 8

_NW = 32
_KP = 4
_CHP = _KP * 128


def _sc_gather_pairs(emb_table, idx_e, idx_o, p):
    per_w = p // _NW
    steps = per_w // _CHP
    mesh = plsc.VectorSubcoreMesh(core_axis_name="c", subcore_axis_name="s")

    @functools.partial(
        pl.kernel,
        mesh=mesh,
        out_type=jax.ShapeDtypeStruct((p, 2 * _EMB), jnp.float32),
        scratch_types=[
            pltpu.VMEM((_KP, 128), jnp.int32),
            pltpu.VMEM((_KP, 128), jnp.int32),
            pltpu.VMEM((_CHP, _EMB), jnp.float32),
            pltpu.VMEM((_CHP, _EMB), jnp.float32),
            pltpu.SemaphoreType.DMA,
        ],
        compiler_params=pltpu.CompilerParams(use_tc_tiling_on_sc=False),
    )
    def gather_kernel(table_hbm, idxe_hbm, idxo_hbm, out_hbm, idxe_v, idxo_v,
                      re_v, ro_v, sem):
        wid = lax.axis_index("s") * 2 + lax.axis_index("c")
        base = wid * per_w

        def body(i, carry):
            p0 = pl.multiple_of(base + i * _CHP, _CHP)
            r0 = pl.multiple_of(p0 // 128, _KP)
            pltpu.sync_copy(idxe_hbm.at[pl.ds(r0, _KP)], idxe_v)
            pltpu.sync_copy(idxo_hbm.at[pl.ds(r0, _KP)], idxo_v)
            copies = []
            for j in range(_KP):
                dst = pl.ds(j * 128, 128)
                copies.append(
                    pltpu.async_copy(table_hbm.at[idxe_v.at[j]],
                                     re_v.at[dst], sem))
                copies.append(
                    pltpu.async_copy(table_hbm.at[idxo_v.at[j]],
                                     ro_v.at[dst], sem))
            for cp in copies:
                cp.wait()
            rows = pl.ds(p0, _CHP)
            pltpu.sync_copy(re_v, out_hbm.at[rows, pl.ds(0, _EMB)])
            pltpu.sync_copy(ro_v, out_hbm.at[rows, pl.ds(_EMB, _EMB)])
            return carry

        lax.fori_loop(0, steps, body, 0)

    return gather_kernel(emb_table, idx_e, idx_o)


_BT = 2048


def _tc_body(t_ref, v_ref, m_ref, g_ref, w1t, b1t, w1v, b1v, w2p, ms_r,
             out_ref):
    w2 = w2p[...]
    ms = ms_r[...]
    dn = (((0,), (0,)), ((), ()))
    podd = lax.broadcasted_iota(jnp.int32, (_EMB, 128), 1) % 2 == 1
    for k in range(_BT // 128):
        row = pl.ds(k, 1)
        xt = t_ref[0, row, :]
        xv = v_ref[0, row, :]
        cm = m_ref[0, row, :].astype(jnp.float32)
        ht = jnp.tanh(w1t[...] * xt + b1t[...])
        hv = jnp.tanh(w1v[...] * xv + b1v[...]) * cm
        h = jnp.concatenate([ht, hv], axis=0)
        pr = g_ref[pl.ds(k * _EMB, _EMB), :]
        lhs = jnp.concatenate([pr, w2], axis=0)
        rhs = jnp.concatenate([ms, h], axis=0)
        res = lax.dot_general(lhs, rhs, dn,
                              preferred_element_type=jnp.float32)
        out_ref[0, :, pl.ds(k * 128, 128)] = jnp.where(
            podd, res[_EMB:, :], res[: _EMB, :])


def _tc_combine(t3, v3, m3, gath, w1t, b1t, w1v, b1v, w2p, ms, l, b):
    bt = b // _BT
    kk = _BT // 128
    return pl.pallas_call(
        _tc_body,
        grid=(l, bt),
        in_specs=[
            pl.BlockSpec((1, kk, 128), lambda i, j: (i, j, 0)),
            pl.BlockSpec((1, kk, 128), lambda i, j: (i, j, 0)),
            pl.BlockSpec((1, kk, 128), lambda i, j: (i, j, 0)),
            pl.BlockSpec((_BT // 2, 128), lambda i, j, bt=bt: (i * bt + j, 0)),
            pl.BlockSpec((_HID, 1), lambda i, j: (0, 0)),
            pl.BlockSpec((_HID, 1), lambda i, j: (0, 0)),
            pl.BlockSpec((_HID, 1), lambda i, j: (0, 0)),
            pl.BlockSpec((_HID, 1), lambda i, j: (0, 0)),
            pl.BlockSpec((2 * _HID, 128), lambda i, j: (0, 0)),
            pl.BlockSpec((_EMB, 128), lambda i, j: (0, 0)),
        ],
        out_specs=pl.BlockSpec((1, _EMB, _BT), lambda i, j: (i, 0, j)),
        out_shape=jax.ShapeDtypeStruct((l, _EMB, b), jnp.float32),
    )(t3, v3, m3, gath, w1t, b1t, w1v, b1v, w2p, ms)


def _pm_body(id_ref, pm_ref):
    pm_ref[...] = jnp.clip(id_ref[...].astype(jnp.float32), 0.0, 1.0)


def _pm_mask(id_t, l, b):
    return pl.pallas_call(
        _pm_body,
        grid=(l // 8,),
        in_specs=[pl.BlockSpec((8, b), lambda i: (i, 0))],
        out_specs=pl.BlockSpec((8, b), lambda i: (i, 0)),
        out_shape=jax.ShapeDtypeStruct((l, b), jnp.float32),
    )(id_t)


def kernel(time, value, var_id, category_mask, W1_t, b1_t, W2_t, W1_v, b1_v,
           W2_v, emb_table):
    b, l = time.shape
    n = b * l
    p = n // 2

    id_t = var_id.T.astype(jnp.int32)
    idx = id_t.reshape(n)
    idx_e = idx[0::2].reshape(p // 128, 128)
    idx_o = idx[1::2].reshape(p // 128, 128)
    gath = _sc_gather_pairs(emb_table, idx_e, idx_o, p)

    cols = jnp.arange(128, dtype=jnp.int32)[None, :]
    rows = jnp.arange(_EMB, dtype=jnp.int32)[:, None]
    ms = (rows == cols // 2).astype(jnp.float32)
    w2cat = jnp.concatenate([W2_t, W2_v], axis=0)
    w2p = jnp.concatenate([w2cat, w2cat], axis=1)

    out3 = _tc_combine(
        time.T.reshape(l, b // 128, 128),
        value.T.reshape(l, b // 128, 128),
        category_mask.T.reshape(l, b // 128, 128).astype(jnp.int32),
        gath,
        W1_t.reshape(_HID, 1),
        b1_t.reshape(_HID, 1),
        W1_v.reshape(_HID, 1),
        b1_v.reshape(_HID, 1),
        w2p,
        ms,
        l,
        b,
    )
    pm = _pm_mask(id_t, l, b)
    return out3.transpose(2, 0, 1), pm.T


# speedup vs baseline: 4.5959x; 1.1264x over previous
import functools

import jax
import jax.numpy as jnp
from jax import lax
from jax.experimental import pallas as pl
from jax.experimental.pallas import tpu as pltpu
from jax.experimental.pallas import tpu_sc as plsc

_EMB = 64
_HID = 8

_NW = 32
_KP = 5
_CHP = _KP * 128
_NS = 4


def _sc_gather_pairs(emb_table, idx_e, idx_o, p):
    per_w = p // _NW
    steps = per_w // _CHP
    mesh = plsc.VectorSubcoreMesh(core_axis_name="c", subcore_axis_name="s")

    @functools.partial(
        pl.kernel,
        mesh=mesh,
        out_type=jax.ShapeDtypeStruct((p, 2 * _EMB), jnp.float32),
        scratch_types=[
            pltpu.VMEM((_KP, 128), jnp.int32),
            pltpu.VMEM((_KP, 128), jnp.int32),
            pltpu.VMEM((_CHP, _EMB), jnp.float32),
            pltpu.VMEM((_CHP, _EMB), jnp.float32),
            pltpu.SemaphoreType.DMA,
        ],
        compiler_params=pltpu.CompilerParams(use_tc_tiling_on_sc=False),
    )
    def gather_kernel(table_hbm, idxe_hbm, idxo_hbm, out_hbm, idxe_v, idxo_v,
                      re_v, ro_v, sem):
        wid = lax.axis_index("s") * 2 + lax.axis_index("c")
        base = wid * per_w

        def body(i, carry):
            p0 = pl.multiple_of(base + i * _CHP, _CHP)
            r0 = pl.multiple_of(p0 // 128, _KP)
            pltpu.sync_copy(idxe_hbm.at[pl.ds(r0, _KP)], idxe_v)
            pltpu.sync_copy(idxo_hbm.at[pl.ds(r0, _KP)], idxo_v)
            copies = []
            for j in range(_KP):
                dst = pl.ds(j * 128, 128)
                copies.append(
                    pltpu.async_copy(table_hbm.at[idxe_v.at[j]],
                                     re_v.at[dst], sem))
                copies.append(
                    pltpu.async_copy(table_hbm.at[idxo_v.at[j]],
                                     ro_v.at[dst], sem))
            for cp in copies:
                cp.wait()
            rows = pl.ds(p0, _CHP)
            pltpu.sync_copy(re_v, out_hbm.at[rows, pl.ds(0, _EMB)])
            pltpu.sync_copy(ro_v, out_hbm.at[rows, pl.ds(_EMB, _EMB)])
            return carry

        lax.fori_loop(0, steps, body, 0)

    return gather_kernel(emb_table, idx_e, idx_o)


_BT = 2048


def _tc_body(t_ref, v_ref, m_ref, g_ref, w1t, b1t, w1v, b1v, w2p, ms_r,
             out_ref):
    w2 = w2p[...]
    ms = ms_r[...]
    dn = (((0,), (0,)), ((), ()))
    podd = lax.broadcasted_iota(jnp.int32, (_EMB, 128), 1) % 2 == 1
    for k in range(_BT // 128):
        row = pl.ds(k, 1)
        xt = t_ref[0, row, :]
        xv = v_ref[0, row, :]
        cm = m_ref[0, row, :].astype(jnp.float32)
        ht = jnp.tanh(w1t[...] * xt + b1t[...])
        hv = jnp.tanh(w1v[...] * xv + b1v[...]) * cm
        h = jnp.concatenate([ht, hv], axis=0)
        pr = g_ref[pl.ds(k * _EMB, _EMB), :]
        lhs = jnp.concatenate([pr, w2], axis=0)
        rhs = jnp.concatenate([ms, h], axis=0)
        res = lax.dot_general(lhs, rhs, dn,
                              preferred_element_type=jnp.float32)
        out_ref[0, :, pl.ds(k * 128, 128)] = jnp.where(
            podd, res[_EMB:, :], res[: _EMB, :])


def _tc_combine_slice(acc, t3, v3, m3, gath_s, w1t, b1t, w1v, b1v, w2p, ms,
                      l, b, l0, ls):
    bt = b // _BT
    kk = _BT // 128
    specs = [
        pl.BlockSpec((1, kk, 128), lambda i, j, l0=l0: (l0 + i, j, 0)),
        pl.BlockSpec((1, kk, 128), lambda i, j, l0=l0: (l0 + i, j, 0)),
        pl.BlockSpec((1, kk, 128), lambda i, j, l0=l0: (l0 + i, j, 0)),
        pl.BlockSpec((_BT // 2, 128), lambda i, j, bt=bt: (i * bt + j, 0)),
        pl.BlockSpec((_HID, 1), lambda i, j: (0, 0)),
        pl.BlockSpec((_HID, 1), lambda i, j: (0, 0)),
        pl.BlockSpec((_HID, 1), lambda i, j: (0, 0)),
        pl.BlockSpec((_HID, 1), lambda i, j: (0, 0)),
        pl.BlockSpec((2 * _HID, 128), lambda i, j: (0, 0)),
        pl.BlockSpec((_EMB, 128), lambda i, j: (0, 0)),
    ]
    args = [t3, v3, m3, gath_s, w1t, b1t, w1v, b1v, w2p, ms]
    kwargs = {}
    body = _tc_body
    if acc is not None:
        specs = [pl.BlockSpec((1, 8, 128), lambda i, j: (0, 0, 0))] + specs
        args = [acc] + args
        kwargs["input_output_aliases"] = {0: 0}
        body = lambda a, *r: _tc_body(*r)
    return pl.pallas_call(
        body,
        grid=(ls, bt),
        in_specs=specs,
        out_specs=pl.BlockSpec((1, _EMB, _BT),
                               lambda i, j, l0=l0: (l0 + i, 0, j)),
        out_shape=jax.ShapeDtypeStruct((l, _EMB, b), jnp.float32),
        **kwargs,
    )(*args)


def _pm_body(id_ref, pm_ref):
    pm_ref[...] = jnp.clip(id_ref[...].astype(jnp.float32), 0.0, 1.0)


def _pm_mask(id_t, l, b):
    return pl.pallas_call(
        _pm_body,
        grid=(l // 8,),
        in_specs=[pl.BlockSpec((8, b), lambda i: (i, 0))],
        out_specs=pl.BlockSpec((8, b), lambda i: (i, 0)),
        out_shape=jax.ShapeDtypeStruct((l, b), jnp.float32),
    )(id_t)


def kernel(time, value, var_id, category_mask, W1_t, b1_t, W2_t, W1_v, b1_v,
           W2_v, emb_table):
    b, l = time.shape
    n = b * l
    p = n // 2

    id_t = var_id.T.astype(jnp.int32)
    idx = id_t.reshape(n)
    idx_e = idx[0::2].reshape(p // 128, 128)
    idx_o = idx[1::2].reshape(p // 128, 128)

    rs = p // 128 // _NS
    gaths = [
        _sc_gather_pairs(emb_table,
                         lax.slice_in_dim(idx_e, s * rs, (s + 1) * rs),
                         lax.slice_in_dim(idx_o, s * rs, (s + 1) * rs),
                         p // _NS)
        for s in range(_NS)
    ]

    cols = jnp.arange(128, dtype=jnp.int32)[None, :]
    rows = jnp.arange(_EMB, dtype=jnp.int32)[:, None]
    ms = (rows == cols // 2).astype(jnp.float32)
    w2cat = jnp.concatenate([W2_t, W2_v], axis=0)
    w2p = jnp.concatenate([w2cat, w2cat], axis=1)

    t3 = time.T.reshape(l, b // 128, 128)
    v3 = value.T.reshape(l, b // 128, 128)
    m3 = category_mask.T.reshape(l, b // 128, 128).astype(jnp.int32)
    w1tc = W1_t.reshape(_HID, 1)
    b1tc = b1_t.reshape(_HID, 1)
    w1vc = W1_v.reshape(_HID, 1)
    b1vc = b1_v.reshape(_HID, 1)

    ls = l // _NS
    out3 = None
    for s in range(_NS):
        out3 = _tc_combine_slice(out3, t3, v3, m3, gaths[s], w1tc, b1tc,
                                 w1vc, b1vc, w2p, ms, l, b, s * ls, ls)
    pm = _pm_mask(id_t, l, b)
    return out3.transpose(2, 0, 1), pm.T


# speedup vs baseline: 5.4721x; 1.1906x over previous
import functools

import jax
import jax.numpy as jnp
from jax import lax
from jax.experimental import pallas as pl
from jax.experimental.pallas import tpu as pltpu
from jax.experimental.pallas import tpu_sc as plsc

_EMB = 64
_HID = 8

_NW = 32
_KP = 5
_CHP = _KP * 128
_NS = 4


def _sc_gather_pairs(emb_table, idx_e, idx_o, p):
    per_w = p // _NW
    steps = per_w // _CHP
    mesh = plsc.VectorSubcoreMesh(core_axis_name="c", subcore_axis_name="s")

    @functools.partial(
        pl.kernel,
        mesh=mesh,
        out_type=jax.ShapeDtypeStruct((p, 2 * _EMB), jnp.float32),
        scratch_types=[
            pltpu.VMEM((_KP, 128), jnp.int32),
            pltpu.VMEM((_KP, 128), jnp.int32),
            pltpu.VMEM((_CHP, _EMB), jnp.float32),
            pltpu.VMEM((_CHP, _EMB), jnp.float32),
            pltpu.SemaphoreType.DMA,
        ],
        compiler_params=pltpu.CompilerParams(use_tc_tiling_on_sc=False),
    )
    def gather_kernel(table_hbm, idxe_hbm, idxo_hbm, out_hbm, idxe_v, idxo_v,
                      re_v, ro_v, sem):
        wid = lax.axis_index("s") * 2 + lax.axis_index("c")
        base = wid * per_w

        def body(i, carry):
            p0 = pl.multiple_of(base + i * _CHP, _CHP)
            r0 = pl.multiple_of(p0 // 128, _KP)
            pltpu.sync_copy(idxe_hbm.at[pl.ds(r0, _KP)], idxe_v)
            pltpu.sync_copy(idxo_hbm.at[pl.ds(r0, _KP)], idxo_v)
            copies = []
            for j in range(_KP):
                dst = pl.ds(j * 128, 128)
                copies.append(
                    pltpu.async_copy(table_hbm.at[idxe_v.at[j]],
                                     re_v.at[dst], sem))
                copies.append(
                    pltpu.async_copy(table_hbm.at[idxo_v.at[j]],
                                     ro_v.at[dst], sem))
            for cp in copies:
                cp.wait()
            rows = pl.ds(p0, _CHP)
            pltpu.sync_copy(re_v, out_hbm.at[rows, pl.ds(0, _EMB)])
            pltpu.sync_copy(ro_v, out_hbm.at[rows, pl.ds(_EMB, _EMB)])
            return carry

        lax.fori_loop(0, steps, body, 0)

    return gather_kernel(emb_table, idx_e, idx_o)


_BT = 4096


def _tc_body(t_ref, v_ref, m_ref, g_ref, w1t, b1t, w1v, b1v, w2p, ms_r,
             out_ref):
    w2 = w2p[...]
    ms = ms_r[...]
    dn = (((0,), (0,)), ((), ()))
    podd = lax.broadcasted_iota(jnp.int32, (_EMB, 128), 1) % 2 == 1
    for k in range(_BT // 128):
        row = pl.ds(k, 1)
        xt = t_ref[0, row, :]
        xv = v_ref[0, row, :]
        cm = m_ref[0, row, :].astype(jnp.float32)
        ht = jnp.tanh(w1t[...] * xt + b1t[...])
        hv = jnp.tanh(w1v[...] * xv + b1v[...]) * cm
        h = jnp.concatenate([ht, hv], axis=0)
        pr = g_ref[pl.ds(k * _EMB, _EMB), :]
        lhs = jnp.concatenate([pr, w2], axis=0)
        rhs = jnp.concatenate([ms, h], axis=0)
        res = lax.dot_general(lhs, rhs, dn,
                              preferred_element_type=jnp.float32)
        out_ref[0, :, pl.ds(k * 128, 128)] = jnp.where(
            podd, res[_EMB:, :], res[: _EMB, :])


def _tc_combine_slice(acc, t3, v3, m3, gath_s, w1t, b1t, w1v, b1v, w2p, ms,
                      l, b, l0, ls):
    bt = b // _BT
    kk = _BT // 128
    specs = [
        pl.BlockSpec((1, kk, 128), lambda i, j, l0=l0: (l0 + i, j, 0)),
        pl.BlockSpec((1, kk, 128), lambda i, j, l0=l0: (l0 + i, j, 0)),
        pl.BlockSpec((1, kk, 128), lambda i, j, l0=l0: (l0 + i, j, 0)),
        pl.BlockSpec((_BT // 2, 128), lambda i, j, bt=bt: (i * bt + j, 0)),
        pl.BlockSpec((_HID, 1), lambda i, j: (0, 0)),
        pl.BlockSpec((_HID, 1), lambda i, j: (0, 0)),
        pl.BlockSpec((_HID, 1), lambda i, j: (0, 0)),
        pl.BlockSpec((_HID, 1), lambda i, j: (0, 0)),
        pl.BlockSpec((2 * _HID, 128), lambda i, j: (0, 0)),
        pl.BlockSpec((_EMB, 128), lambda i, j: (0, 0)),
    ]
    args = [t3, v3, m3, gath_s, w1t, b1t, w1v, b1v, w2p, ms]
    kwargs = {}
    body = _tc_body
    if acc is not None:
        specs = [pl.BlockSpec((1, 8, 128), lambda i, j: (0, 0, 0))] + specs
        args = [acc] + args
        kwargs["input_output_aliases"] = {0: 0}
        body = lambda a, *r: _tc_body(*r)
    return pl.pallas_call(
        body,
        grid=(ls, bt),
        in_specs=specs,
        out_specs=pl.BlockSpec((1, _EMB, _BT),
                               lambda i, j, l0=l0: (l0 + i, 0, j)),
        out_shape=jax.ShapeDtypeStruct((l, _EMB, b), jnp.float32),
        **kwargs,
    )(*args)


def _pm_body(id_ref, pm_ref):
    pm_ref[...] = jnp.clip(id_ref[...].astype(jnp.float32), 0.0, 1.0)


def _pm_mask(id_t, l, b):
    return pl.pallas_call(
        _pm_body,
        grid=(l // 8,),
        in_specs=[pl.BlockSpec((8, b), lambda i: (i, 0))],
        out_specs=pl.BlockSpec((8, b), lambda i: (i, 0)),
        out_shape=jax.ShapeDtypeStruct((l, b), jnp.float32),
    )(id_t)


def kernel(time, value, var_id, category_mask, W1_t, b1_t, W2_t, W1_v, b1_v,
           W2_v, emb_table):
    b, l = time.shape
    n = b * l
    p = n // 2

    id_t = var_id.T.astype(jnp.int32)
    idx = id_t.reshape(n)
    idx_e = idx[0::2].reshape(p // 128, 128)
    idx_o = idx[1::2].reshape(p // 128, 128)

    rs = p // 128 // _NS
    gaths = [
        _sc_gather_pairs(emb_table,
                         lax.slice_in_dim(idx_e, s * rs, (s + 1) * rs),
                         lax.slice_in_dim(idx_o, s * rs, (s + 1) * rs),
                         p // _NS)
        for s in range(_NS)
    ]

    cols = jnp.arange(128, dtype=jnp.int32)[None, :]
    rows = jnp.arange(_EMB, dtype=jnp.int32)[:, None]
    ms = (rows == cols // 2).astype(jnp.float32)
    w2cat = jnp.concatenate([W2_t, W2_v], axis=0)
    w2p = jnp.concatenate([w2cat, w2cat], axis=1)

    t3 = time.T.reshape(l, b // 128, 128)
    v3 = value.T.reshape(l, b // 128, 128)
    m3 = category_mask.T.reshape(l, b // 128, 128).astype(jnp.int32)
    w1tc = W1_t.reshape(_HID, 1)
    b1tc = b1_t.reshape(_HID, 1)
    w1vc = W1_v.reshape(_HID, 1)
    b1vc = b1_v.reshape(_HID, 1)

    ls = l // _NS
    out3 = None
    for s in range(_NS):
        out3 = _tc_combine_slice(out3, t3, v3, m3, gaths[s], w1tc, b1tc,
                                 w1vc, b1vc, w2p, ms, l, b, s * ls, ls)
    pm = _pm_mask(id_t, l, b)
    return out3.transpose(2, 0, 1), pm.T


# speedup vs baseline: 6.9302x; 1.2665x over previous
import functools

import jax
import jax.numpy as jnp
from jax import lax
from jax.experimental import pallas as pl
from jax.experimental.pallas import tpu as pltpu
from jax.experimental.pallas import tpu_sc as plsc

_EMB = 64
_HID = 8

_NW = 32
_KP = 5
_CHP = _KP * 128
_NS = 4


def _sc_gather_pairs(emb_table, idx2, p, row_off):
    per_w = p // _NW
    steps = per_w // _CHP
    mesh = plsc.VectorSubcoreMesh(core_axis_name="c", subcore_axis_name="s")

    @functools.partial(
        pl.kernel,
        mesh=mesh,
        out_type=jax.ShapeDtypeStruct((p, 2 * _EMB), jnp.float32),
        scratch_types=[
            pltpu.VMEM((2 * _KP, 128), jnp.int32),
            pltpu.VMEM((_KP, 128), jnp.int32),
            pltpu.VMEM((_KP, 128), jnp.int32),
            pltpu.VMEM((_CHP, _EMB), jnp.float32),
            pltpu.VMEM((_CHP, _EMB), jnp.float32),
            pltpu.SemaphoreType.DMA,
        ],
        compiler_params=pltpu.CompilerParams(use_tc_tiling_on_sc=False,
                                             needs_layout_passes=False),
    )
    def gather_kernel(table_hbm, idx_hbm, out_hbm, idxr_v, idxe_v, idxo_v,
                      re_v, ro_v, sem):
        wid = lax.axis_index("s") * 2 + lax.axis_index("c")
        base = wid * per_w
        iota = lax.broadcasted_iota(jnp.int32, (16,), 0)

        def body(i, carry):
            p0 = pl.multiple_of(base + i * _CHP, _CHP)
            r0 = pl.multiple_of(row_off + p0 // 64, 2 * _KP)
            pltpu.sync_copy(idx_hbm.at[pl.ds(r0, 2 * _KP)], idxr_v)
            for k in range(_KP * 8):
                rows16 = jnp.full((16,), k // 4, jnp.int32)
                cols = (k % 4) * 32 + 2 * iota
                ev = plsc.load_gather(idxr_v, [rows16, cols])
                od = plsc.load_gather(idxr_v, [rows16, cols + 1])
                dst = pl.ds((16 * k) % 128, 16)
                idxe_v[k // 8, dst] = ev
                idxo_v[k // 8, dst] = od
            copies = []
            for j in range(_KP):
                dst = pl.ds(j * 128, 128)
                copies.append(
                    pltpu.async_copy(table_hbm.at[idxe_v.at[j]],
                                     re_v.at[dst], sem))
                copies.append(
                    pltpu.async_copy(table_hbm.at[idxo_v.at[j]],
                                     ro_v.at[dst], sem))
            for cp in copies:
                cp.wait()
            rows = pl.ds(p0, _CHP)
            pltpu.sync_copy(re_v, out_hbm.at[rows, pl.ds(0, _EMB)])
            pltpu.sync_copy(ro_v, out_hbm.at[rows, pl.ds(_EMB, _EMB)])
            return carry

        lax.fori_loop(0, steps, body, 0)

    return gather_kernel(emb_table, idx2)


_BT = 4096


def _tc_body(t_ref, v_ref, m_ref, g_ref, w1t, b1t, w1v, b1v, w2p, ms_r,
             out_ref):
    w2 = w2p[...]
    ms = ms_r[...]
    dn = (((0,), (0,)), ((), ()))
    podd = lax.broadcasted_iota(jnp.int32, (_EMB, 128), 1) % 2 == 1
    for k in range(_BT // 128):
        row = pl.ds(k, 1)
        xt = t_ref[0, row, :]
        xv = v_ref[0, row, :]
        cm = m_ref[0, row, :].astype(jnp.float32)
        ht = jnp.tanh(w1t[...] * xt + b1t[...])
        hv = jnp.tanh(w1v[...] * xv + b1v[...]) * cm
        h = jnp.concatenate([ht, hv], axis=0)
        pr = g_ref[pl.ds(k * _EMB, _EMB), :]
        lhs = jnp.concatenate([pr, w2], axis=0)
        rhs = jnp.concatenate([ms, h], axis=0)
        res = lax.dot_general(lhs, rhs, dn,
                              preferred_element_type=jnp.float32)
        out_ref[0, :, pl.ds(k * 128, 128)] = jnp.where(
            podd, res[_EMB:, :], res[: _EMB, :])


def _tc_combine_slice(acc, t3, v3, m3, gath_s, w1t, b1t, w1v, b1v, w2p, ms,
                      l, b, l0, ls):
    bt = b // _BT
    kk = _BT // 128
    specs = [
        pl.BlockSpec((1, kk, 128), lambda i, j, l0=l0: (l0 + i, j, 0)),
        pl.BlockSpec((1, kk, 128), lambda i, j, l0=l0: (l0 + i, j, 0)),
        pl.BlockSpec((1, kk, 128), lambda i, j, l0=l0: (l0 + i, j, 0)),
        pl.BlockSpec((_BT // 2, 128), lambda i, j, bt=bt: (i * bt + j, 0)),
        pl.BlockSpec((_HID, 1), lambda i, j: (0, 0)),
        pl.BlockSpec((_HID, 1), lambda i, j: (0, 0)),
        pl.BlockSpec((_HID, 1), lambda i, j: (0, 0)),
        pl.BlockSpec((_HID, 1), lambda i, j: (0, 0)),
        pl.BlockSpec((2 * _HID, 128), lambda i, j: (0, 0)),
        pl.BlockSpec((_EMB, 128), lambda i, j: (0, 0)),
    ]
    args = [t3, v3, m3, gath_s, w1t, b1t, w1v, b1v, w2p, ms]
    kwargs = {}
    body = _tc_body
    if acc is not None:
        specs = [pl.BlockSpec((1, 8, 128), lambda i, j: (0, 0, 0))] + specs
        args = [acc] + args
        kwargs["input_output_aliases"] = {0: 0}
        body = lambda a, *r: _tc_body(*r)
    return pl.pallas_call(
        body,
        grid=(ls, bt),
        in_specs=specs,
        out_specs=pl.BlockSpec((1, _EMB, _BT),
                               lambda i, j, l0=l0: (l0 + i, 0, j)),
        out_shape=jax.ShapeDtypeStruct((l, _EMB, b), jnp.float32),
        **kwargs,
    )(*args)


def _pm_body(id_ref, pm_ref):
    pm_ref[...] = jnp.clip(id_ref[...].astype(jnp.float32), 0.0, 1.0)


def _pm_mask(id_t, l, b):
    return pl.pallas_call(
        _pm_body,
        grid=(l // 8,),
        in_specs=[pl.BlockSpec((8, b), lambda i: (i, 0))],
        out_specs=pl.BlockSpec((8, b), lambda i: (i, 0)),
        out_shape=jax.ShapeDtypeStruct((l, b), jnp.float32),
    )(id_t)


def kernel(time, value, var_id, category_mask, W1_t, b1_t, W2_t, W1_v, b1_v,
           W2_v, emb_table):
    b, l = time.shape
    n = b * l
    p = n // 2

    id_t = var_id.T.astype(jnp.int32)
    idx2 = id_t.reshape(n // 128, 128)

    rs = n // 128 // _NS
    gaths = [
        _sc_gather_pairs(emb_table, idx2, p // _NS, s * rs)
        for s in range(_NS)
    ]

    cols = jnp.arange(128, dtype=jnp.int32)[None, :]
    rows = jnp.arange(_EMB, dtype=jnp.int32)[:, None]
    ms = (rows == cols // 2).astype(jnp.float32)
    w2cat = jnp.concatenate([W2_t, W2_v], axis=0)
    w2p = jnp.concatenate([w2cat, w2cat], axis=1)

    t3 = time.T.reshape(l, b // 128, 128)
    v3 = value.T.reshape(l, b // 128, 128)
    m3 = category_mask.T.reshape(l, b // 128, 128).astype(jnp.int32)
    w1tc = W1_t.reshape(_HID, 1)
    b1tc = b1_t.reshape(_HID, 1)
    w1vc = W1_v.reshape(_HID, 1)
    b1vc = b1_v.reshape(_HID, 1)

    ls = l // _NS
    out3 = None
    for s in range(_NS):
        out3 = _tc_combine_slice(out3, t3, v3, m3, gaths[s], w1tc, b1tc,
                                 w1vc, b1vc, w2p, ms, l, b, s * ls, ls)
    pm = _pm_mask(id_t, l, b)
    return out3.transpose(2, 0, 1), pm.T
